# Initial kernel scaffold; baseline (speedup 1.0000x reference)
import jax
import jax.numpy as jnp
from jax.experimental import pallas as pl


def kernel(x, sample_map, output_shape):
    raise NotImplementedError("write your pallas kernel here")


# speedup vs baseline: 10.6588x; 10.6588x over previous
import functools

import jax
import jax.numpy as jnp
from jax import lax
from jax.experimental import pallas as pl
from jax.experimental.pallas import tpu as pltpu
from jax.experimental.pallas import tpu_sc as plsc

H = 512
W = 512
P = H * W
BC = 192
NCORES = 2
NSUB = 16
PLANES_PER_CORE = BC // NCORES
PT = P // NSUB
CHUNK = 4096
NCHUNK = PT // CHUNK
ROWS = CHUNK // 128

_CORNERS = ((0, 0), (1, 0), (0, 1), (1, 1))


def _corner_kernel(smx_ref, smy_ref, idx_ref, w_ref):
    sx = smx_ref[...]
    sy = smy_ref[...]
    x0 = jnp.floor(sx)
    y0 = jnp.floor(sy)
    fx = sx - x0
    fy = sy - y0
    x0i = x0.astype(jnp.int32)
    y0i = y0.astype(jnp.int32)
    for k, (dx, dy) in enumerate(_CORNERS):
        xi = x0i + dx
        yi = y0i + dy
        valid = (xi >= 0) & (xi < W) & (yi >= 0) & (yi < H)
        idx = jnp.clip(yi, 0, H - 1) * W + jnp.clip(xi, 0, W - 1)
        wx = fx if dx else 1.0 - fx
        wy = fy if dy else 1.0 - fy
        idx_ref[k] = idx
        w_ref[k] = jnp.where(valid, wx * wy, 0.0)


def _compute_corners(smx, smy):
    return pl.pallas_call(
        _corner_kernel,
        out_shape=[
            jax.ShapeDtypeStruct((4, H, W), jnp.int32),
            jax.ShapeDtypeStruct((4, H, W), jnp.float32),
        ],
    )(smx, smy)


def _sc_body(x_hbm, idx_hbm, w_hbm, out_hbm, acc, xb, ib, wb, vb, zb):
    cid = lax.axis_index("c")
    sid = lax.axis_index("s")

    def zinit(i, carry):
        zb[pl.ds(pl.multiple_of(i * 16, 16), 16)] = jnp.zeros((16,), jnp.float32)
        return carry

    lax.fori_loop(0, CHUNK // 16, zinit, 0)

    def plane_body(pi, carry):
        plane = cid * PLANES_PER_CORE + pi
        base_t = sid * PT
        for c in range(NCHUNK):
            pltpu.sync_copy(
                zb, acc.at[pl.ds(pl.multiple_of(base_t + c * CHUNK, CHUNK), CHUNK)]
            )
        plsc.subcore_barrier()
        for c in range(NCHUNK):
            base = pl.multiple_of(base_t + c * CHUNK, CHUNK)
            pltpu.sync_copy(x_hbm.at[plane, pl.ds(base, CHUNK)], xb)
            row0 = pl.multiple_of(sid * (PT // 128) + c * ROWS, ROWS)
            for k in range(4):
                pltpu.sync_copy(idx_hbm.at[k, pl.ds(row0, ROWS)], ib.at[k])
                pltpu.sync_copy(w_hbm.at[k, pl.ds(base, CHUNK)], wb.at[k])
            for k in range(4):

                def mul(i, carry, k=k):
                    s = pl.ds(pl.multiple_of(i * 16, 16), 16)
                    vb[s] = xb[s] * wb[k, s]
                    return carry

                lax.fori_loop(0, CHUNK // 16, mul, 0)

                def scat(j, carry, k=k):
                    pltpu.sync_copy(
                        vb.at[pl.ds(pl.multiple_of(j * 128, 128), 128)],
                        acc.at[ib.at[k, j]],
                        add=True,
                    )
                    return carry

                lax.fori_loop(0, ROWS, scat, 0)
        plsc.subcore_barrier()
        pltpu.sync_copy(
            acc.at[pl.ds(pl.multiple_of(base_t, CHUNK), PT)],
            out_hbm.at[plane, pl.ds(pl.multiple_of(base_t, CHUNK), PT)],
        )
        plsc.subcore_barrier()
        return carry

    lax.fori_loop(0, PLANES_PER_CORE, plane_body, 0)


def _sc_scatter(x2d, idx4, w4):
    mesh = plsc.VectorSubcoreMesh(
        core_axis_name="c", subcore_axis_name="s", num_cores=NCORES
    )
    fn = pl.kernel(
        _sc_body,
        out_type=jax.ShapeDtypeStruct((BC, P), jnp.float32),
        mesh=mesh,
        scratch_types=[
            pltpu.VMEM_SHARED((P,), jnp.float32),
            pltpu.VMEM((CHUNK,), jnp.float32),
            pltpu.VMEM((4, ROWS, 128), jnp.int32),
            pltpu.VMEM((4, CHUNK), jnp.float32),
            pltpu.VMEM((CHUNK,), jnp.float32),
            pltpu.VMEM((CHUNK,), jnp.float32),
        ],
    )
    return fn(x2d, idx4, w4)


@jax.jit
def kernel(x, sample_map, output_shape):
    del output_shape
    B, C, Hin, Win = x.shape
    smx = sample_map[..., 0]
    smy = sample_map[..., 1]
    idx4, w4 = _compute_corners(smx, smy)
    idx4 = idx4.reshape(4, P // 128, 128)
    w4 = w4.reshape(4, P)
    x2d = x.reshape(B * C, Hin * Win)
    out = _sc_scatter(x2d, idx4, w4)
    return out.reshape(B, C, H, W)


# speedup vs baseline: 14.6941x; 1.3786x over previous
import functools

import jax
import jax.numpy as jnp
from jax import lax
from jax.experimental import pallas as pl
from jax.experimental.pallas import tpu as pltpu
from jax.experimental.pallas import tpu_sc as plsc

H = 512
W = 512
P = H * W
BC = 192
G = 4
NCORES = 2
NSUB = 16
GROUPS_PER_CORE = BC // (NCORES * G)
PT = P // NSUB
CH = 1024
NCHUNK = PT // CH
IROWS = 2 * CH // 128
NFIRE = IROWS
ZROWS = 512


def _corner_kernel(smx_ref, smy_ref, base_ref, w_ref):
    sx = smx_ref[...]
    sy = smy_ref[...]
    x0 = jnp.floor(sx)
    y0 = jnp.floor(sy)
    x0i = jnp.clip(x0.astype(jnp.int32), 0, W - 2)
    y0i = jnp.clip(y0.astype(jnp.int32), 0, H - 2)
    fx = sx - x0i.astype(jnp.float32)
    fy = sy - y0i.astype(jnp.float32)
    base_ref[...] = y0i * W + x0i
    w_ref[0] = (1.0 - fx) * (1.0 - fy)
    w_ref[1] = fx * (1.0 - fy)
    w_ref[2] = (1.0 - fx) * fy
    w_ref[3] = fx * fy


def _compute_corners(smx, smy):
    return pl.pallas_call(
        _corner_kernel,
        out_shape=[
            jax.ShapeDtypeStruct((H, W), jnp.int32),
            jax.ShapeDtypeStruct((4, H, W), jnp.float32),
        ],
    )(smx, smy)


def _sc_body(x_hbm, base_hbm, w_hbm, out_hbm, acc, xb, wb, bflat, idxq, vals, wstage, zb, sem):
    cid = lax.axis_index("c")
    sid = lax.axis_index("s")
    iota = lax.iota(jnp.int32, 16)
    half = (iota >> 2) & 1
    colv = iota & 7
    civ = iota & 3
    unit2 = iota >> 3
    zero16 = jnp.zeros((16,), jnp.float32)
    base_t = sid * PT

    def z_init(i, c):
        zb[i, pl.ds(0, 8)] = jnp.zeros((8,), jnp.float32)
        return c

    def z_init16(i, c):
        plsc.store_scatter(zb, [i * 2 + unit2, colv], zero16)
        return c

    lax.fori_loop(0, ZROWS // 2, z_init16, 0)
    for z in range(PT // 2 // ZROWS):
        pltpu.sync_copy(
            zb, acc.at[pl.ds(pl.multiple_of(base_t // 2 + z * ZROWS, ZROWS), ZROWS)]
        )
    plsc.subcore_barrier()

    def group_body(g, carry):
        pg = cid * (GROUPS_PER_CORE * G) + g * G

        def chunk_body(cc, carry2):
            base = pl.multiple_of(base_t + cc * CH, CH)
            for gp in range(G):
                pltpu.sync_copy(x_hbm.at[pg + gp, pl.ds(base, CH)], xb.at[gp])
            for k in range(4):
                pltpu.sync_copy(w_hbm.at[k, pl.ds(base, CH)], wb.at[k])
            pltpu.sync_copy(base_hbm.at[pl.ds(base, CH)], bflat)

            for l in range(2):
                lw = l * W

                def d_body(j, c3, l=l):
                    pltpu.make_async_copy(
                        vals.at[l, pl.ds(pl.multiple_of(j * 128, 128), 128)],
                        acc.at[idxq.at[l, j]],
                        sem,
                    ).wait()
                    return c3

                @pl.when(cc > 0)
                def _drain():
                    lax.fori_loop(0, NFIRE, d_body, 0)

                def i_body(r, c3, l=l, lw=lw):
                    for v in range(8):
                        s = pl.ds(pl.multiple_of(v * 16, 16), 16)
                        bv = bflat[pl.ds(pl.multiple_of(r * 128 + v * 16, 16), 16)] + lw
                        idxq[l, r, s] = bv >> 1
                        idxq[l, (CH // 128) + r, s] = (bv + 1) >> 1
                    return c3

                lax.fori_loop(0, CH // 128, i_body, 0)

                def v_body(i, c3, l=l, lw=lw):
                    pixv = i * 2 + unit2
                    xg = plsc.load_gather(xb, [civ, pixv])
                    wa = plsc.load_gather(wb, [jnp.full((16,), 2 * l, jnp.int32), pixv])
                    wbv = plsc.load_gather(
                        wb, [jnp.full((16,), 2 * l + 1, jnp.int32), pixv]
                    )
                    bv = plsc.load_gather(bflat, [pixv]) + lw
                    side = bv & 1
                    va = jnp.where(half == side, xg * wa, 0.0)
                    vb2 = jnp.where(half == (side ^ 1), xg * wbv, 0.0)
                    plsc.store_scatter(vals.at[l], [pixv, colv], va)
                    plsc.store_scatter(vals.at[l], [CH + pixv, colv], vb2)
                    return c3

                lax.fori_loop(0, CH // 2, v_body, 0)

                def f_body(j, c3, l=l):
                    pltpu.async_copy(
                        vals.at[l, pl.ds(pl.multiple_of(j * 128, 128), 128)],
                        acc.at[idxq.at[l, j]],
                        sem,
                        add=True,
                    )
                    return c3

                lax.fori_loop(0, NFIRE, f_body, 0)
            return carry2

        lax.fori_loop(0, NCHUNK, chunk_body, 0)

        for l in range(2):
            def d_body(j, c3, l=l):
                pltpu.make_async_copy(
                    vals.at[l, pl.ds(pl.multiple_of(j * 128, 128), 128)],
                    acc.at[idxq.at[l, j]],
                    sem,
                ).wait()
                return c3

            lax.fori_loop(0, NFIRE, d_body, 0)
        plsc.subcore_barrier()

        def wb_body(cc, carry2):
            c0 = pl.multiple_of(base_t + cc * CH, CH)
            r0 = pl.multiple_of(base_t // 2 + cc * (CH // 2), CH // 2)
            pltpu.sync_copy(acc.at[pl.ds(r0, CH // 2)], vals.at[0, pl.ds(0, CH // 2)])
            pltpu.sync_copy(zb.at[pl.ds(0, CH // 2)], acc.at[pl.ds(r0, CH // 2)])

            def t_body(i, c3):
                cv = i * 16 + iota
                rowv = cv >> 1
                colb = (cv & 1) << 2
                for gp in range(G):
                    v = plsc.load_gather(vals.at[0], [rowv, colb + gp])
                    wstage[gp, pl.ds(pl.multiple_of(i * 16, 16), 16)] = v
                return c3

            lax.fori_loop(0, CH // 16, t_body, 0)
            for gp in range(G):
                pltpu.sync_copy(wstage.at[gp], out_hbm.at[pg + gp, pl.ds(c0, CH)])
            return carry2

        lax.fori_loop(0, NCHUNK, wb_body, 0)
        plsc.subcore_barrier()
        return carry

    lax.fori_loop(0, GROUPS_PER_CORE, group_body, 0)


def _sc_scatter(x2d, base_flat, w4):
    mesh = plsc.VectorSubcoreMesh(
        core_axis_name="c", subcore_axis_name="s", num_cores=NCORES, num_subcores=NSUB
    )
    fn = pl.kernel(
        _sc_body,
        out_type=jax.ShapeDtypeStruct((BC, P), jnp.float32),
        mesh=mesh,
        scratch_types=[
            pltpu.VMEM_SHARED((P // 2, 8), jnp.float32),
            pltpu.VMEM((G, CH), jnp.float32),
            pltpu.VMEM((4, CH), jnp.float32),
            pltpu.VMEM((CH,), jnp.int32),
            pltpu.VMEM((2, IROWS, 128), jnp.int32),
            pltpu.VMEM((2, 2 * CH, 8), jnp.float32),
            pltpu.VMEM((G, CH), jnp.float32),
            pltpu.VMEM((ZROWS, 8), jnp.float32),
            pltpu.SemaphoreType.DMA,
        ],
        compiler_params=pltpu.CompilerParams(
            use_tc_tiling_on_sc=False, needs_layout_passes=False
        ),
    )
    return fn(x2d, base_flat, w4)


@jax.jit
def kernel(x, sample_map, output_shape):
    del output_shape
    B, C, Hin, Win = x.shape
    smx = sample_map[..., 0]
    smy = sample_map[..., 1]
    base_arr, w4 = _compute_corners(smx, smy)
    base_flat = base_arr.reshape(P)
    w4 = w4.reshape(4, P)
    x2d = x.reshape(B * C, Hin * Win)
    out = _sc_scatter(x2d, base_flat, w4)
    return out.reshape(B, C, H, W)


# speedup vs baseline: 18.0134x; 1.2259x over previous
import functools

import jax
import jax.numpy as jnp
from jax import lax
from jax.experimental import pallas as pl
from jax.experimental.pallas import tpu as pltpu
from jax.experimental.pallas import tpu_sc as plsc

H = 512
W = 512
P = H * W
BC = 192
G = 4
NCORES = 2
NSUB = 16
GROUPS_PER_CORE = BC // (NCORES * G)
PT = P // NSUB
CH = 1024
NCHUNK = PT // CH
IROWS = 2 * CH // 128
NFIRE = IROWS
ZROWS = 256


def _corner_kernel(smx_ref, smy_ref, base_ref, w_ref):
    sx = smx_ref[...]
    sy = smy_ref[...]
    x0 = jnp.floor(sx)
    y0 = jnp.floor(sy)
    x0i = jnp.clip(x0.astype(jnp.int32), 0, W - 2)
    y0i = jnp.clip(y0.astype(jnp.int32), 0, H - 2)
    fx = sx - x0i.astype(jnp.float32)
    fy = sy - y0i.astype(jnp.float32)
    base_ref[...] = y0i * W + x0i
    w_ref[0] = (1.0 - fx) * (1.0 - fy)
    w_ref[1] = fx * (1.0 - fy)
    w_ref[2] = (1.0 - fx) * fy
    w_ref[3] = fx * fy


def _compute_corners(smx, smy):
    return pl.pallas_call(
        _corner_kernel,
        out_shape=[
            jax.ShapeDtypeStruct((H, W), jnp.int32),
            jax.ShapeDtypeStruct((4, H, W), jnp.float32),
        ],
    )(smx, smy)


def _sc_body(
    x_hbm, base_hbm, w_hbm, out_hbm, acc, xb, wb, bflat, idxq, vals, wstage, zb, sem, sem2
):
    cid = lax.axis_index("c")
    sid = lax.axis_index("s")
    iota = lax.iota(jnp.int32, 16)
    half = (iota >> 2) & 1
    colv = iota & 7
    civ = iota & 3
    unit2 = iota >> 3
    zero16 = jnp.zeros((16,), jnp.float32)
    base_t = sid * PT

    def z_init(i, c):
        zb[i, pl.ds(0, 8)] = jnp.zeros((8,), jnp.float32)
        return c

    def z_init16(i, c):
        plsc.store_scatter(zb, [i * 2 + unit2, colv], zero16)
        return c

    lax.fori_loop(0, ZROWS // 2, z_init16, 0)
    for z in range(PT // 2 // ZROWS):
        pltpu.sync_copy(
            zb, acc.at[pl.ds(pl.multiple_of(base_t // 2 + z * ZROWS, ZROWS), ZROWS)]
        )
    plsc.subcore_barrier()

    def group_body(g, carry):
        pg = cid * (GROUPS_PER_CORE * G) + g * G

        def chunk_body(cc, carry2):
            base = pl.multiple_of(base_t + cc * CH, CH)
            for gp in range(G):
                pltpu.async_copy(x_hbm.at[pg + gp, pl.ds(base, CH)], xb.at[gp], sem2)
            for k in range(4):
                pltpu.async_copy(w_hbm.at[k, pl.ds(base, CH)], wb.at[k], sem2)
            pltpu.sync_copy(base_hbm.at[pl.ds(base, CH)], bflat)

            for l in range(2):
                lw = l * W

                def d_body(j, c3, l=l):
                    pltpu.make_async_copy(
                        vals.at[l, pl.ds(pl.multiple_of(j * 128, 128), 128)],
                        acc.at[idxq.at[l, j]],
                        sem,
                    ).wait()
                    return c3

                @pl.when(cc > 0)
                def _drain():
                    lax.fori_loop(0, NFIRE, d_body, 0)

                def i_body(r, c3, l=l, lw=lw):
                    for v in range(8):
                        s = pl.ds(pl.multiple_of(v * 16, 16), 16)
                        bv = bflat[pl.ds(pl.multiple_of(r * 128 + v * 16, 16), 16)] + lw
                        idxq[l, r, s] = bv >> 1
                        idxq[l, (CH // 128) + r, s] = (bv + 1) >> 1
                    return c3

                lax.fori_loop(0, CH // 128, i_body, 0)

                if l == 0:
                    for gp in range(G):
                        pltpu.make_async_copy(
                            x_hbm.at[pg + gp, pl.ds(base, CH)], xb.at[gp], sem2
                        ).wait()
                    for k in range(4):
                        pltpu.make_async_copy(
                            w_hbm.at[k, pl.ds(base, CH)], wb.at[k], sem2
                        ).wait()

                wka = jnp.full((16,), 2 * l, jnp.int32)
                wkb = jnp.full((16,), 2 * l + 1, jnp.int32)

                def v_body(i, c3, l=l, lw=lw, wka=wka, wkb=wkb):
                    for u in range(4):
                        pixv = i * 8 + u * 2 + unit2
                        xg = plsc.load_gather(xb, [civ, pixv])
                        wa = plsc.load_gather(wb, [wka, pixv])
                        wbv = plsc.load_gather(wb, [wkb, pixv])
                        bv = plsc.load_gather(bflat, [pixv]) + lw
                        side = bv & 1
                        va = jnp.where(half == side, xg * wa, 0.0)
                        vb2 = jnp.where(half == (side ^ 1), xg * wbv, 0.0)
                        plsc.store_scatter(vals.at[l], [pixv, colv], va)
                        plsc.store_scatter(vals.at[l], [CH + pixv, colv], vb2)
                    return c3

                lax.fori_loop(0, CH // 8, v_body, 0)

                def f_body(j, c3, l=l):
                    pltpu.async_copy(
                        vals.at[l, pl.ds(pl.multiple_of(j * 128, 128), 128)],
                        acc.at[idxq.at[l, j]],
                        sem,
                        add=True,
                    )
                    return c3

                lax.fori_loop(0, NFIRE, f_body, 0)
            return carry2

        lax.fori_loop(0, NCHUNK, chunk_body, 0)

        for l in range(2):
            def d_body(j, c3, l=l):
                pltpu.make_async_copy(
                    vals.at[l, pl.ds(pl.multiple_of(j * 128, 128), 128)],
                    acc.at[idxq.at[l, j]],
                    sem,
                ).wait()
                return c3

            lax.fori_loop(0, NFIRE, d_body, 0)
        plsc.subcore_barrier()

        def wb_body(cc, carry2):
            c0 = pl.multiple_of(base_t + cc * CH, CH)
            r0 = pl.multiple_of(base_t // 2 + cc * (CH // 2), CH // 2)
            pltpu.sync_copy(acc.at[pl.ds(r0, CH // 2)], vals.at[0, pl.ds(0, CH // 2)])
            for z in range(CH // 2 // ZROWS):
                pltpu.sync_copy(
                    zb, acc.at[pl.ds(pl.multiple_of(r0 + z * ZROWS, ZROWS), ZROWS)]
                )

            def t_body(i, c3):
                for u in range(2):
                    cv = (i * 2 + u) * 16 + iota
                    rowv = cv >> 1
                    colb = (cv & 1) << 2
                    for gp in range(G):
                        v = plsc.load_gather(vals.at[0], [rowv, colb + gp])
                        wstage[gp, pl.ds(pl.multiple_of((i * 2 + u) * 16, 16), 16)] = v
                return c3

            lax.fori_loop(0, CH // 32, t_body, 0)
            for gp in range(G):
                pltpu.sync_copy(wstage.at[gp], out_hbm.at[pg + gp, pl.ds(c0, CH)])
            return carry2

        lax.fori_loop(0, NCHUNK, wb_body, 0)
        plsc.subcore_barrier()
        return carry

    lax.fori_loop(0, GROUPS_PER_CORE, group_body, 0)


def _sc_scatter(x2d, base_flat, w4):
    mesh = plsc.VectorSubcoreMesh(
        core_axis_name="c", subcore_axis_name="s", num_cores=NCORES, num_subcores=NSUB
    )
    fn = pl.kernel(
        _sc_body,
        out_type=jax.ShapeDtypeStruct((BC, P), jnp.float32),
        mesh=mesh,
        scratch_types=[
            pltpu.VMEM_SHARED((P // 2, 8), jnp.float32),
            pltpu.VMEM((G, CH), jnp.float32),
            pltpu.VMEM((4, CH), jnp.float32),
            pltpu.VMEM((CH,), jnp.int32),
            pltpu.VMEM((2, IROWS, 128), jnp.int32),
            pltpu.VMEM((2, 2 * CH, 8), jnp.float32),
            pltpu.VMEM((G, CH), jnp.float32),
            pltpu.VMEM((ZROWS, 8), jnp.float32),
            pltpu.SemaphoreType.DMA,
            pltpu.SemaphoreType.DMA,
        ],
        compiler_params=pltpu.CompilerParams(
            use_tc_tiling_on_sc=False, needs_layout_passes=False
        ),
    )
    return fn(x2d, base_flat, w4)


@jax.jit
def kernel(x, sample_map, output_shape):
    del output_shape
    B, C, Hin, Win = x.shape
    smx = sample_map[..., 0]
    smy = sample_map[..., 1]
    base_arr, w4 = _compute_corners(smx, smy)
    base_flat = base_arr.reshape(P)
    w4 = w4.reshape(4, P)
    x2d = x.reshape(B * C, Hin * Win)
    out = _sc_scatter(x2d, base_flat, w4)
    return out.reshape(B, C, H, W)


# speedup vs baseline: 37.9062x; 2.1043x over previous
import functools

import jax
import jax.numpy as jnp
from jax import lax
from jax.experimental import pallas as pl
from jax.experimental.pallas import tpu as pltpu
from jax.experimental.pallas import tpu_sc as plsc

H = 512
W = 512
P = H * W
BC = 192
G = 4
NCORES = 2
NSUB = 16
GROUPS_PER_CORE = BC // (NCORES * G)
PT = P // NSUB
CH = 1024
NCHUNK = PT // CH
IROWS = 2 * CH // 128
NFIRE = IROWS
ZROWS = 256


def _corner_kernel(smx_ref, smy_ref, base_ref, w_ref):
    sx = smx_ref[...]
    sy = smy_ref[...]
    x0 = jnp.floor(sx)
    y0 = jnp.floor(sy)
    x0i = jnp.clip(x0.astype(jnp.int32), 0, W - 2)
    y0i = jnp.clip(y0.astype(jnp.int32), 0, H - 2)
    fx = sx - x0i.astype(jnp.float32)
    fy = sy - y0i.astype(jnp.float32)
    base_ref[...] = y0i * W + x0i
    w_ref[0] = (1.0 - fx) * (1.0 - fy)
    w_ref[1] = fx * (1.0 - fy)
    w_ref[2] = (1.0 - fx) * fy
    w_ref[3] = fx * fy


def _compute_corners(smx, smy):
    return pl.pallas_call(
        _corner_kernel,
        out_shape=[
            jax.ShapeDtypeStruct((H, W), jnp.int32),
            jax.ShapeDtypeStruct((4, H, W), jnp.float32),
        ],
    )(smx, smy)


def _sc_body(
    x_hbm, base_hbm, w_hbm, out_hbm, acc, xb, wb, bflat, idxq, vals, wstage, zb, sem, sem2
):
    cid = lax.axis_index("c")
    sid = lax.axis_index("s")
    iota = lax.iota(jnp.int32, 16)
    half = (iota >> 2) & 1
    colv = iota & 7
    civ = iota & 3
    unit2 = iota >> 3
    zero16 = jnp.zeros((16,), jnp.float32)
    base_t = sid * PT

    def z_init(i, c):
        zb[i, pl.ds(0, 8)] = jnp.zeros((8,), jnp.float32)
        return c

    def z_init16(i, c):
        plsc.store_scatter(zb, [i * 2 + unit2, colv], zero16)
        return c

    lax.fori_loop(0, ZROWS // 2, z_init16, 0)
    for z in range(PT // 2 // ZROWS):
        pltpu.sync_copy(
            zb, acc.at[pl.ds(pl.multiple_of(base_t // 2 + z * ZROWS, ZROWS), ZROWS)]
        )

    def zb2_init(i, c):
        rowv = CH + i * 2 + unit2
        for l in range(2):
            plsc.store_scatter(vals.at[l], [rowv, 4 + civ], zero16)
        return c

    lax.fori_loop(0, CH // 2, zb2_init, 0)
    plsc.subcore_barrier()

    def group_body(g, carry):
        pg = cid * (GROUPS_PER_CORE * G) + g * G

        def chunk_body(cc, carry2):
            base = pl.multiple_of(base_t + cc * CH, CH)
            for gp in range(G):
                pltpu.async_copy(x_hbm.at[pg + gp, pl.ds(base, CH)], xb.at[gp], sem2)
            for k in range(4):
                pltpu.async_copy(w_hbm.at[k, pl.ds(base, CH)], wb.at[k], sem2)
            pltpu.sync_copy(base_hbm.at[pl.ds(base, CH)], bflat)

            for l in range(2):
                lw = l * W

                def d_body(j, c3, l=l):
                    pltpu.make_async_copy(
                        vals.at[l, pl.ds(pl.multiple_of(j * 128, 128), 128)],
                        acc.at[idxq.at[l, j]],
                        sem,
                    ).wait()
                    return c3

                @pl.when(cc > 0)
                def _drain():
                    lax.fori_loop(0, NFIRE, d_body, 0)

                def i_body(r, c3, l=l, lw=lw):
                    for v in range(8):
                        s = pl.ds(pl.multiple_of(v * 16, 16), 16)
                        bv = bflat[pl.ds(pl.multiple_of(r * 128 + v * 16, 16), 16)] + lw
                        idxq[l, r, s] = bv >> 1
                        idxq[l, (CH // 128) + r, s] = (bv + 1) >> 1
                    return c3

                lax.fori_loop(0, CH // 128, i_body, 0)

                if l == 0:
                    for gp in range(G):
                        pltpu.make_async_copy(
                            x_hbm.at[pg + gp, pl.ds(base, CH)], xb.at[gp], sem2
                        ).wait()
                    for k in range(4):
                        pltpu.make_async_copy(
                            w_hbm.at[k, pl.ds(base, CH)], wb.at[k], sem2
                        ).wait()

                def v_body(i, c3, l=l, lw=lw):
                    s = pl.ds(pl.multiple_of(i * 16, 16), 16)
                    pix16 = i * 16 + iota
                    even = ((bflat[s] + lw) & 1) == 0
                    wa16 = wb[2 * l, s]
                    wb16 = wb[2 * l + 1, s]
                    rowA = pix16
                    rowB = CH + pix16
                    for gp in range(G):
                        xv = xb[gp, s]
                        pa = xv * wa16
                        pb = xv * wb16
                        cg = jnp.full((16,), gp, jnp.int32)
                        plsc.store_scatter(
                            vals.at[l], [rowA, cg], jnp.where(even, pa, 0.0)
                        )
                        plsc.store_scatter(
                            vals.at[l], [rowA, cg + 4], jnp.where(even, pb, pa)
                        )
                        plsc.store_scatter(
                            vals.at[l], [rowB, cg], jnp.where(even, 0.0, pb)
                        )
                    return c3

                lax.fori_loop(0, CH // 16, v_body, 0)

                def f_body(j, c3, l=l):
                    pltpu.async_copy(
                        vals.at[l, pl.ds(pl.multiple_of(j * 128, 128), 128)],
                        acc.at[idxq.at[l, j]],
                        sem,
                        add=True,
                    )
                    return c3

                lax.fori_loop(0, NFIRE, f_body, 0)
            return carry2

        lax.fori_loop(0, NCHUNK, chunk_body, 0)

        for l in range(2):
            def d_body(j, c3, l=l):
                pltpu.make_async_copy(
                    vals.at[l, pl.ds(pl.multiple_of(j * 128, 128), 128)],
                    acc.at[idxq.at[l, j]],
                    sem,
                ).wait()
                return c3

            lax.fori_loop(0, NFIRE, d_body, 0)
        plsc.subcore_barrier()

        def wb_body(cc, carry2):
            c0 = pl.multiple_of(base_t + cc * CH, CH)
            r0 = pl.multiple_of(base_t // 2 + cc * (CH // 2), CH // 2)
            pltpu.sync_copy(acc.at[pl.ds(r0, CH // 2)], vals.at[0, pl.ds(0, CH // 2)])
            for z in range(CH // 2 // ZROWS):
                pltpu.sync_copy(
                    zb, acc.at[pl.ds(pl.multiple_of(r0 + z * ZROWS, ZROWS), ZROWS)]
                )

            def t_body(i, c3):
                for u in range(2):
                    cv = (i * 2 + u) * 16 + iota
                    rowv = cv >> 1
                    colb = (cv & 1) << 2
                    for gp in range(G):
                        v = plsc.load_gather(vals.at[0], [rowv, colb + gp])
                        wstage[gp, pl.ds(pl.multiple_of((i * 2 + u) * 16, 16), 16)] = v
                return c3

            lax.fori_loop(0, CH // 32, t_body, 0)
            for gp in range(G):
                pltpu.sync_copy(wstage.at[gp], out_hbm.at[pg + gp, pl.ds(c0, CH)])
            return carry2

        lax.fori_loop(0, NCHUNK, wb_body, 0)
        plsc.subcore_barrier()
        return carry

    lax.fori_loop(0, GROUPS_PER_CORE, group_body, 0)


def _sc_scatter(x2d, base_flat, w4):
    mesh = plsc.VectorSubcoreMesh(
        core_axis_name="c", subcore_axis_name="s", num_cores=NCORES, num_subcores=NSUB
    )
    fn = pl.kernel(
        _sc_body,
        out_type=jax.ShapeDtypeStruct((BC, P), jnp.float32),
        mesh=mesh,
        scratch_types=[
            pltpu.VMEM_SHARED((P // 2, 8), jnp.float32),
            pltpu.VMEM((G, CH), jnp.float32),
            pltpu.VMEM((4, CH), jnp.float32),
            pltpu.VMEM((CH,), jnp.int32),
            pltpu.VMEM((2, IROWS, 128), jnp.int32),
            pltpu.VMEM((2, 2 * CH, 8), jnp.float32),
            pltpu.VMEM((G, CH), jnp.float32),
            pltpu.VMEM((ZROWS, 8), jnp.float32),
            pltpu.SemaphoreType.DMA,
            pltpu.SemaphoreType.DMA,
        ],
        compiler_params=pltpu.CompilerParams(
            use_tc_tiling_on_sc=False, needs_layout_passes=False
        ),
    )
    return fn(x2d, base_flat, w4)


@jax.jit
def kernel(x, sample_map, output_shape):
    del output_shape
    B, C, Hin, Win = x.shape
    smx = sample_map[..., 0]
    smy = sample_map[..., 1]
    base_arr, w4 = _compute_corners(smx, smy)
    base_flat = base_arr.reshape(P)
    w4 = w4.reshape(4, P)
    x2d = x.reshape(B * C, Hin * Win)
    out = _sc_scatter(x2d, base_flat, w4)
    return out.reshape(B, C, H, W)


# speedup vs baseline: 38.0031x; 1.0026x over previous
import functools

import jax
import jax.numpy as jnp
from jax import lax
from jax.experimental import pallas as pl
from jax.experimental.pallas import tpu as pltpu
from jax.experimental.pallas import tpu_sc as plsc

H = 512
W = 512
P = H * W
BC = 192
G = 4
NCORES = 2
NSUB = 16
GROUPS_PER_CORE = BC // (NCORES * G)
PT = P // NSUB
CH = 1024
NCHUNK = PT // CH
IROWS = 2 * CH // 128
NFIRE = IROWS
ZROWS = 256


def _corner_kernel(smx_ref, smy_ref, base_ref, w_ref):
    sx = smx_ref[...]
    sy = smy_ref[...]
    x0 = jnp.floor(sx)
    y0 = jnp.floor(sy)
    x0i = jnp.clip(x0.astype(jnp.int32), 0, W - 2)
    y0i = jnp.clip(y0.astype(jnp.int32), 0, H - 2)
    fx = sx - x0i.astype(jnp.float32)
    fy = sy - y0i.astype(jnp.float32)
    base_ref[...] = y0i * W + x0i
    w_ref[0] = (1.0 - fx) * (1.0 - fy)
    w_ref[1] = fx * (1.0 - fy)
    w_ref[2] = (1.0 - fx) * fy
    w_ref[3] = fx * fy


def _compute_corners(smx, smy):
    return pl.pallas_call(
        _corner_kernel,
        out_shape=[
            jax.ShapeDtypeStruct((H, W), jnp.int32),
            jax.ShapeDtypeStruct((4, H, W), jnp.float32),
        ],
    )(smx, smy)


def _sc_body(
    x_hbm, base_hbm, w_hbm, out_hbm, acc, xb, wb, bflat, idxq, vals, wstage, zb, sem, sem2
):
    cid = lax.axis_index("c")
    sid = lax.axis_index("s")
    iota = lax.iota(jnp.int32, 16)
    half = (iota >> 2) & 1
    colv = iota & 7
    civ = iota & 3
    unit2 = iota >> 3
    zero16 = jnp.zeros((16,), jnp.float32)
    base_t = sid * PT

    def z_init(i, c):
        zb[i, pl.ds(0, 8)] = jnp.zeros((8,), jnp.float32)
        return c

    def z_init16(i, c):
        plsc.store_scatter(zb, [i * 2 + unit2, colv], zero16)
        return c

    lax.fori_loop(0, ZROWS // 2, z_init16, 0)
    for z in range(PT // 2 // ZROWS):
        pltpu.sync_copy(
            zb, acc.at[pl.ds(pl.multiple_of(base_t // 2 + z * ZROWS, ZROWS), ZROWS)]
        )

    def zb2_init(i, c):
        rowv = CH + i * 2 + unit2
        for l in range(2):
            plsc.store_scatter(vals.at[l], [rowv, 4 + civ], zero16)
        return c

    lax.fori_loop(0, CH // 2, zb2_init, 0)
    plsc.subcore_barrier()

    def group_body(g, carry):
        pg = cid * (GROUPS_PER_CORE * G) + g * G

        def chunk_body(cc, carry2):
            base = pl.multiple_of(base_t + cc * CH, CH)
            for gp in range(G):
                pltpu.async_copy(x_hbm.at[pg + gp, pl.ds(base, CH)], xb.at[gp], sem2)
            for k in range(4):
                pltpu.async_copy(w_hbm.at[k, pl.ds(base, CH)], wb.at[k], sem2)
            pltpu.sync_copy(base_hbm.at[pl.ds(base, CH)], bflat)

            for l in range(2):
                lw = l * W

                def d_body(j, c3, l=l):
                    pltpu.make_async_copy(
                        vals.at[l, pl.ds(pl.multiple_of(j * 128, 128), 128)],
                        acc.at[idxq.at[l, j]],
                        sem,
                    ).wait()
                    return c3

                @pl.when(cc > 0)
                def _drain():
                    lax.fori_loop(0, NFIRE, d_body, 0)

                def i_body(r, c3, l=l, lw=lw):
                    for v in range(8):
                        s = pl.ds(pl.multiple_of(v * 16, 16), 16)
                        bv = bflat[pl.ds(pl.multiple_of(r * 128 + v * 16, 16), 16)] + lw
                        idxq[l, r, s] = bv >> 1
                        idxq[l, (CH // 128) + r, s] = (bv + 1) >> 1
                    return c3

                lax.fori_loop(0, CH // 128, i_body, 0)

                if l == 0:
                    for gp in range(G):
                        pltpu.make_async_copy(
                            x_hbm.at[pg + gp, pl.ds(base, CH)], xb.at[gp], sem2
                        ).wait()
                    for k in range(4):
                        pltpu.make_async_copy(
                            w_hbm.at[k, pl.ds(base, CH)], wb.at[k], sem2
                        ).wait()

                def v_body(i, c3, l=l, lw=lw):
                    for u in range(2):
                        ii = i * 2 + u
                        s = pl.ds(pl.multiple_of(ii * 16, 16), 16)
                        pix16 = ii * 16 + iota
                        even = ((bflat[s] + lw) & 1) == 0
                        wa16 = wb[2 * l, s]
                        wb16 = wb[2 * l + 1, s]
                        rowA = pix16
                        rowB = CH + pix16
                        for gp in range(G):
                            xv = xb[gp, s]
                            pa = xv * wa16
                            pb = xv * wb16
                            cg = jnp.full((16,), gp, jnp.int32)
                            plsc.store_scatter(
                                vals.at[l], [rowA, cg], jnp.where(even, pa, 0.0)
                            )
                            plsc.store_scatter(
                                vals.at[l], [rowA, cg + 4], jnp.where(even, pb, pa)
                            )
                            plsc.store_scatter(
                                vals.at[l], [rowB, cg], jnp.where(even, 0.0, pb)
                            )
                    return c3

                lax.fori_loop(0, CH // 32, v_body, 0)

                def f_body(j, c3, l=l):
                    pltpu.async_copy(
                        vals.at[l, pl.ds(pl.multiple_of(j * 128, 128), 128)],
                        acc.at[idxq.at[l, j]],
                        sem,
                        add=True,
                    )
                    return c3

                lax.fori_loop(0, NFIRE, f_body, 0)
            return carry2

        lax.fori_loop(0, NCHUNK, chunk_body, 0)

        for l in range(2):
            def d_body(j, c3, l=l):
                pltpu.make_async_copy(
                    vals.at[l, pl.ds(pl.multiple_of(j * 128, 128), 128)],
                    acc.at[idxq.at[l, j]],
                    sem,
                ).wait()
                return c3

            lax.fori_loop(0, NFIRE, d_body, 0)
        plsc.subcore_barrier()

        def wb_body(cc, carry2):
            c0 = pl.multiple_of(base_t + cc * CH, CH)
            r0 = pl.multiple_of(base_t // 2 + cc * (CH // 2), CH // 2)
            pltpu.sync_copy(acc.at[pl.ds(r0, CH // 2)], vals.at[0, pl.ds(0, CH // 2)])
            for z in range(CH // 2 // ZROWS):
                pltpu.sync_copy(
                    zb, acc.at[pl.ds(pl.multiple_of(r0 + z * ZROWS, ZROWS), ZROWS)]
                )

            def t_body(i, c3):
                for u in range(2):
                    cv = (i * 2 + u) * 16 + iota
                    rowv = cv >> 1
                    colb = (cv & 1) << 2
                    for gp in range(G):
                        v = plsc.load_gather(vals.at[0], [rowv, colb + gp])
                        wstage[gp, pl.ds(pl.multiple_of((i * 2 + u) * 16, 16), 16)] = v
                return c3

            lax.fori_loop(0, CH // 32, t_body, 0)
            for gp in range(G):
                pltpu.sync_copy(wstage.at[gp], out_hbm.at[pg + gp, pl.ds(c0, CH)])
            return carry2

        lax.fori_loop(0, NCHUNK, wb_body, 0)
        plsc.subcore_barrier()
        return carry

    lax.fori_loop(0, GROUPS_PER_CORE, group_body, 0)


def _sc_scatter(x2d, base_flat, w4):
    mesh = plsc.VectorSubcoreMesh(
        core_axis_name="c", subcore_axis_name="s", num_cores=NCORES, num_subcores=NSUB
    )
    fn = pl.kernel(
        _sc_body,
        out_type=jax.ShapeDtypeStruct((BC, P), jnp.float32),
        mesh=mesh,
        scratch_types=[
            pltpu.VMEM_SHARED((P // 2, 8), jnp.float32),
            pltpu.VMEM((G, CH), jnp.float32),
            pltpu.VMEM((4, CH), jnp.float32),
            pltpu.VMEM((CH,), jnp.int32),
            pltpu.VMEM((2, IROWS, 128), jnp.int32),
            pltpu.VMEM((2, 2 * CH, 8), jnp.float32),
            pltpu.VMEM((G, CH), jnp.float32),
            pltpu.VMEM((ZROWS, 8), jnp.float32),
            pltpu.SemaphoreType.DMA,
            pltpu.SemaphoreType.DMA,
        ],
        compiler_params=pltpu.CompilerParams(
            use_tc_tiling_on_sc=False, needs_layout_passes=False
        ),
    )
    return fn(x2d, base_flat, w4)


@jax.jit
def kernel(x, sample_map, output_shape):
    del output_shape
    B, C, Hin, Win = x.shape
    smx = sample_map[..., 0]
    smy = sample_map[..., 1]
    base_arr, w4 = _compute_corners(smx, smy)
    base_flat = base_arr.reshape(P)
    w4 = w4.reshape(4, P)
    x2d = x.reshape(B * C, Hin * Win)
    out = _sc_scatter(x2d, base_flat, w4)
    return out.reshape(B, C, H, W)
